# Initial kernel scaffold; baseline (speedup 1.0000x reference)
import jax
import jax.numpy as jnp
from jax.experimental import pallas as pl


def kernel(wordids, lens, W):
    raise NotImplementedError("write your pallas kernel here")


# speedup vs baseline: 1.6198x; 1.6198x over previous
import functools

import jax
import jax.numpy as jnp
from jax import lax
from jax.experimental import pallas as pl
from jax.experimental.pallas import tpu as pltpu
from jax.experimental.pallas import tpu_sc as plsc

B, L, V, D = 4096, 200, 1000000, 32
NC, NS = 2, 16
NW = NC * NS
RPW = B // NW
LN = 16
NCH = 13
PADL = NCH * LN


def _body(wid_hbm, lens_hbm, w_hbm, out_hbm,
          wid_v, lensb_v, rows0, rows1, out_v, w0_v, lens_sc, sem0, sem1):
    cid = lax.axis_index("c")
    sid = lax.axis_index("s")
    w = cid * NS + sid
    base = w * RPW

    pltpu.sync_copy(wid_hbm.at[pl.ds(base, RPW), :], wid_v.at[:, :])
    pltpu.sync_copy(w_hbm.at[pl.ds(0, 1), :], w0_v)

    iotaf = lax.iota(jnp.int32, LN).astype(jnp.float32)

    pltpu.sync_copy(lens_hbm.at[pl.ds(base, RPW)], lens_sc)
    lane = lax.iota(jnp.int32, LN)

    def lens_group(g, _):
        lv = lens_sc[pl.ds(g * LN, LN)].astype(jnp.float32)

        def lens_row(r2, _):
            lsc = jnp.max(jnp.where(lane == r2, lv, 0.0))
            lensb_v[g * LN + r2, :] = jnp.full((LN,), 1.0, jnp.float32) * lsc
            return 0

        lax.fori_loop(0, LN, lens_row, 0)
        return 0

    lax.fori_loop(0, RPW // LN, lens_group, 0)

    def nchunks(lfb):
        li = jnp.max(lfb).astype(jnp.int32)
        return (li + (LN - 1)) // LN

    def fire(r, rows_buf, sem):
        lfb = lensb_v[r, :]
        nch = nchunks(lfb)

        def fire_chunk(c, _):
            off = jnp.minimum(c * LN, L - LN)
            lowf = (c * LN).astype(jnp.float32)
            posf = iotaf + off.astype(jnp.float32)
            ids = wid_v[r, pl.ds(off, LN)]
            ids = jnp.where((posf >= lowf) & (posf < lfb), ids, 0)
            pltpu.async_copy(w_hbm.at[ids], rows_buf.at[pl.ds(c * LN, LN), :], sem)
            return 0

        lax.fori_loop(0, nch, fire_chunk, 0)

    def drain(r, rows_buf, sem):
        lfb = lensb_v[r, :]
        nch = nchunks(lfb)
        zeros_i = jnp.zeros((LN,), jnp.int32)

        def drain_chunk(c, _):
            pltpu.make_async_copy(
                w_hbm.at[zeros_i], rows_buf.at[pl.ds(0, LN), :], sem).wait()
            return 0

        lax.fori_loop(0, nch, drain_chunk, 0)

    def accum(r, rows_buf):
        lfb = lensb_v[r, :]
        nch = nchunks(lfb)
        zero = jnp.zeros((LN,), jnp.float32)

        def chunk_body(c, accs):
            accs = list(accs)
            b16 = c * LN
            for u in range(LN):
                accs[2 * (u % 4)] = accs[2 * (u % 4)] + rows_buf[b16 + u, pl.ds(0, LN)]
                accs[2 * (u % 4) + 1] = accs[2 * (u % 4) + 1] + rows_buf[b16 + u, pl.ds(LN, LN)]
            return tuple(accs)

        accs = lax.fori_loop(0, nch, chunk_body, (zero,) * 8)
        acc0 = (accs[0] + accs[2]) + (accs[4] + accs[6])
        acc1 = (accs[1] + accs[3]) + (accs[5] + accs[7])
        zf = nch.astype(jnp.float32) * float(LN) - lfb
        w0a = w0_v[0, pl.ds(0, LN)]
        w0b = w0_v[0, pl.ds(LN, LN)]
        out_v[r, pl.ds(0, LN)] = (acc0 - zf * w0a) / lfb
        out_v[r, pl.ds(LN, LN)] = (acc1 - zf * w0b) / lfb

    fire(0, rows0, sem0)

    def outer(k, _):
        r0 = 2 * k
        fire(r0 + 1, rows1, sem1)
        drain(r0, rows0, sem0)
        accum(r0, rows0)

        @pl.when(k < RPW // 2 - 1)
        def _():
            fire(r0 + 2, rows0, sem0)

        drain(r0 + 1, rows1, sem1)
        accum(r0 + 1, rows1)
        return 0

    lax.fori_loop(0, RPW // 2, outer, 0)
    pltpu.sync_copy(out_v.at[:, :], out_hbm.at[pl.ds(base, RPW), :])


@jax.jit
def kernel(wordids, lens, W):
    mesh = plsc.VectorSubcoreMesh(core_axis_name="c", subcore_axis_name="s")
    f = functools.partial(
        pl.kernel,
        out_type=jax.ShapeDtypeStruct((B, D), jnp.float32),
        mesh=mesh,
        compiler_params=pltpu.CompilerParams(
            needs_layout_passes=False, use_tc_tiling_on_sc=False),
        scratch_types=[
            pltpu.VMEM((RPW, L), jnp.int32),
            pltpu.VMEM((RPW, LN), jnp.float32),
            pltpu.VMEM((PADL, D), jnp.float32),
            pltpu.VMEM((PADL, D), jnp.float32),
            pltpu.VMEM((RPW, D), jnp.float32),
            pltpu.VMEM((1, D), jnp.float32),
            pltpu.VMEM((RPW,), jnp.int32),
            pltpu.SemaphoreType.DMA,
            pltpu.SemaphoreType.DMA,
        ],
    )(_body)
    return f(wordids, lens, W)
